# baseline (speedup 1.0000x reference)
import functools

import jax
import jax.numpy as jnp
from jax import lax
from jax.experimental import pallas as pl
from jax.experimental.pallas import tpu as pltpu, tpu_sc as plsc

N_NODES = 100000
MAXDEG = 64
SAMPLES1 = 25
SAMPLES2 = 10
BATCH = 4096

NUM_WORKERS = 32
B0_PER_W = BATCH // NUM_WORKERS
B1_PER_W = B0_PER_W * SAMPLES2
N_CHUNKS = SAMPLES2
CHUNK = B0_PER_W
OUT1_CHUNK = CHUNK * SAMPLES1


def _sampler_kernel(x, adj_table, p0, p1):
  mesh = plsc.VectorSubcoreMesh(core_axis_name="c", subcore_axis_name="s")

  @functools.partial(
      pl.kernel,
      out_type=(
          jax.ShapeDtypeStruct((NUM_WORKERS, SAMPLES2, CHUNK), jnp.int32),
          jax.ShapeDtypeStruct((NUM_WORKERS * N_CHUNKS * OUT1_CHUNK,),
                               jnp.int32),
      ),
      mesh=mesh,
      compiler_params=pltpu.CompilerParams(
          needs_layout_passes=False, use_tc_tiling_on_sc=False),
      scratch_types=[
          pltpu.VMEM((B0_PER_W,), jnp.int32),
          pltpu.VMEM((CHUNK, MAXDEG), jnp.int32),
          pltpu.VMEM((SAMPLES2, CHUNK), jnp.int32),
          pltpu.VMEM((OUT1_CHUNK,), jnp.int32),
          pltpu.VMEM((16,), jnp.int32),
          pltpu.VMEM((32,), jnp.int32),
          pltpu.SemaphoreType.DMA,
      ],
  )
  def body(x_h, adj_h, p0_h, p1_h, out0_h, out1_h,
           idx0_v, rows_v, out0_v, out1c_v, p0_v, p1_v, sem):
    wid = lax.axis_index("s") * 2 + lax.axis_index("c")
    lane = lax.iota(jnp.int32, 16)
    mask10 = lane < SAMPLES2
    mask9 = lane < (SAMPLES1 - 16)

    pltpu.sync_copy(p0_h, p0_v)
    pltpu.sync_copy(p1_h, p1_v)
    pltpu.sync_copy(x_h.at[pl.ds(wid * B0_PER_W, B0_PER_W)], idx0_v)
    pltpu.async_copy(adj_h.at[idx0_v], rows_v, sem).wait()

    p0vec = p0_v[...]

    def sel0(r, carry):
      rv = jnp.full((16,), r, jnp.int32)
      vals = plsc.load_gather(rows_v, [rv, p0vec])
      f = r * SAMPLES2 + lane
      plsc.store_scatter(
          out0_v,
          [lax.shift_right_logical(f, 7), lax.bitwise_and(f, 127)],
          vals, mask=mask10)
      return carry

    lax.fori_loop(0, B0_PER_W, sel0, 0)
    pltpu.sync_copy(out0_v, out0_h.at[wid])

    p1a = p1_v[pl.ds(0, 16)]
    p1b = p1_v[pl.ds(16, 16)]

    for c in range(N_CHUNKS):
      pltpu.async_copy(adj_h.at[out0_v.at[c]], rows_v, sem).wait()

      def sel1(r, carry):
        rv = jnp.full((16,), r, jnp.int32)
        va = plsc.load_gather(rows_v, [rv, p1a])
        vb = plsc.load_gather(rows_v, [rv, p1b])
        f = r * SAMPLES1 + lane
        plsc.store_scatter(out1c_v, [f], va)
        plsc.store_scatter(out1c_v, [f + 16], vb, mask=mask9)
        return carry

      lax.fori_loop(0, CHUNK, sel1, 0)
      pltpu.sync_copy(
          out1c_v,
          out1_h.at[pl.ds(wid * N_CHUNKS * OUT1_CHUNK + c * OUT1_CHUNK,
                          OUT1_CHUNK)])

  return body(x, adj_table, p0, p1)


def kernel(x, adj_table):
  key = jax.random.key(42)
  k0, k1 = jax.random.split(key)
  perm0 = jax.random.permutation(k0, MAXDEG).astype(jnp.int32)[:SAMPLES2]
  perm1 = jax.random.permutation(k1, MAXDEG).astype(jnp.int32)[:SAMPLES1]
  p0 = jnp.concatenate([perm0, jnp.zeros((16 - SAMPLES2,), jnp.int32)])
  p1 = jnp.concatenate([perm1, jnp.zeros((32 - SAMPLES1,), jnp.int32)])

  out0, out1 = _sampler_kernel(x, adj_table, p0, p1)
  hop0 = out0.reshape(BATCH, SAMPLES2)
  hop1 = out1.reshape(BATCH * SAMPLES2, SAMPLES1)
  return (hop0, hop1)


# speedup vs baseline: 1.3461x; 1.3461x over previous
import functools

import jax
import jax.numpy as jnp
import numpy as np
from jax import lax
from jax.experimental import pallas as pl
from jax.experimental.pallas import tpu as pltpu, tpu_sc as plsc

N_NODES = 100000
MAXDEG = 64
SAMPLES1 = 25
SAMPLES2 = 10
BATCH = 4096

NUM_WORKERS = 32
B0_PER_W = BATCH // NUM_WORKERS
N_CHUNKS = SAMPLES2
CHUNK = B0_PER_W
RGROUPS = CHUNK // 16


_PERM0 = [17, 27, 42, 32, 1, 3, 58, 51, 40, 28]
_PERM1 = [2, 32, 15, 10, 48, 25, 28, 0, 49, 4, 60, 42, 21, 11, 20,
          57, 17, 12, 19, 22, 18, 16, 27, 5, 23]


def _sampler_kernel(x, adj_r):
  mesh = plsc.VectorSubcoreMesh(
      core_axis_name="c", subcore_axis_name="s", num_cores=2, num_subcores=16)

  @functools.partial(
      pl.kernel,
      out_type=(
          jax.ShapeDtypeStruct((SAMPLES2, BATCH), jnp.int32),
          jax.ShapeDtypeStruct((SAMPLES1, BATCH * SAMPLES2), jnp.int32),
      ),
      mesh=mesh,
      compiler_params=pltpu.CompilerParams(
          needs_layout_passes=False, use_tc_tiling_on_sc=True),
      scratch_types=[
          pltpu.VMEM((B0_PER_W,), jnp.int32),
          pltpu.VMEM((B0_PER_W,), jnp.int32),
          pltpu.VMEM((B0_PER_W,), jnp.int32),
          pltpu.VMEM((CHUNK, 2 * MAXDEG), jnp.int32),
          pltpu.VMEM((CHUNK, 2 * MAXDEG), jnp.int32),
          pltpu.VMEM((SAMPLES2, CHUNK), jnp.int32),
          pltpu.VMEM((SAMPLES2, CHUNK), jnp.int32),
          pltpu.VMEM((SAMPLES2, CHUNK), jnp.int32),
          pltpu.VMEM((SAMPLES1, CHUNK), jnp.int32),
          pltpu.VMEM((SAMPLES1, CHUNK), jnp.int32),
          pltpu.SemaphoreType.DMA,
          pltpu.SemaphoreType.DMA,
          pltpu.SemaphoreType.DMA,
          pltpu.SemaphoreType.DMA,
      ],
  )
  def body(x_h, adj_h, out0_h, out1_h,
           xs_v, idx0h_v, par0_v, rows0_v, rows1_v, out0t_v, idx1h_v, par1_v,
           stg0_v, stg1_v, gsem0, gsem1, osem0, osem1):
    wid = lax.axis_index("s") * 2 + lax.axis_index("c")
    lane = lax.iota(jnp.int32, 16)
    rowss = [rows0_v, rows1_v]
    stgs = [stg0_v, stg1_v]
    gsems = [gsem0, gsem1]
    osems = [osem0, osem1]

    pltpu.sync_copy(x_h.at[pl.ds(wid * B0_PER_W, B0_PER_W)], xs_v)
    for k in range(RGROUPS):
      v = xs_v[pl.ds(k * 16, 16)]
      idx0h_v[pl.ds(k * 16, 16)] = lax.shift_right_logical(v, 1)
      par0_v[pl.ds(k * 16, 16)] = lax.shift_left(lax.bitwise_and(v, 1), 6)
    pltpu.async_copy(adj_h.at[idx0h_v], rows0_v, gsem0).wait()

    def sel0(rb, carry):
      base = rb * 16
      rvec = base + lane
      parb = par0_v[pl.ds(base, 16)]
      for j in range(SAMPLES2):
        vals = plsc.load_gather(rows0_v, [rvec, parb + _PERM0[j]])
        out0t_v[j, pl.ds(base, 16)] = vals
        f = rvec * SAMPLES2 + j
        fd = lax.shift_right_logical(f, 7)
        fm = lax.bitwise_and(f, 127)
        plsc.store_scatter(idx1h_v, [fd, fm], lax.shift_right_logical(vals, 1))
        plsc.store_scatter(par1_v, [fd, fm],
                           lax.shift_left(lax.bitwise_and(vals, 1), 6))
      return carry

    lax.fori_loop(0, RGROUPS, sel0, 0)
    pltpu.async_copy(out0t_v, out0_h.at[:, pl.ds(wid * B0_PER_W, B0_PER_W)],
                     osem0).wait()

    out_copies = [None, None]
    gather_copies = [None, None]
    gather_copies[0] = pltpu.async_copy(
        adj_h.at[idx1h_v.at[0]], rows0_v, gsem0)
    for c in range(N_CHUNKS):
      b = c % 2
      if c + 1 < N_CHUNKS:
        gather_copies[(c + 1) % 2] = pltpu.async_copy(
            adj_h.at[idx1h_v.at[c + 1]], rowss[(c + 1) % 2],
            gsems[(c + 1) % 2])
      gather_copies[b].wait()
      if out_copies[b] is not None:
        out_copies[b].wait()
      rows_b, stg_b = rowss[b], stgs[b]

      def sel1(rb, carry):
        base = rb * 16
        rvec = base + lane
        parb = par1_v[c, pl.ds(base, 16)]
        for j in range(SAMPLES1):
          vals = plsc.load_gather(rows_b, [rvec, parb + _PERM1[j]])
          stg_b[j, pl.ds(base, 16)] = vals
        return carry

      lax.fori_loop(0, RGROUPS, sel1, 0)
      out_copies[b] = pltpu.async_copy(
          stgs[b],
          out1_h.at[:, pl.ds(wid * N_CHUNKS * CHUNK + c * CHUNK, CHUNK)],
          osems[b])
    out_copies[0].wait()
    out_copies[1].wait()

  return body(x, adj_r)


def kernel(x, adj_table):
  adj_r = adj_table.reshape(N_NODES // 2, 2 * MAXDEG)
  out0t, out1t = _sampler_kernel(x, adj_r)
  return (out0t.T, out1t.T)


# speedup vs baseline: 1.4661x; 1.0892x over previous
import functools

import jax
import jax.numpy as jnp
from jax import lax
from jax.experimental import pallas as pl
from jax.experimental.pallas import tpu as pltpu, tpu_sc as plsc

N_NODES = 100000
MAXDEG = 64
SAMPLES1 = 25
SAMPLES2 = 10
BATCH = 4096

NUM_WORKERS = 32
B0_PER_W = BATCH // NUM_WORKERS
N_CHUNKS = SAMPLES2
CHUNK = B0_PER_W
RGROUPS = CHUNK // 16

_PERM0 = [17, 27, 42, 32, 1, 3, 58, 51, 40, 28]
_PERM1 = [2, 32, 15, 10, 48, 25, 28, 0, 49, 4, 60, 42, 21, 11, 20,
          57, 17, 12, 19, 22, 18, 16, 27, 5, 23]
_PAD = MAXDEG


def _sampler_kernel(x, adj_p):
  mesh = plsc.VectorSubcoreMesh(
      core_axis_name="c", subcore_axis_name="s", num_cores=2, num_subcores=16)

  @functools.partial(
      pl.kernel,
      out_type=(
          jax.ShapeDtypeStruct((SAMPLES2, BATCH), jnp.int32),
          jax.ShapeDtypeStruct((SAMPLES1, BATCH * SAMPLES2), jnp.int32),
      ),
      mesh=mesh,
      compiler_params=pltpu.CompilerParams(
          needs_layout_passes=False, use_tc_tiling_on_sc=True),
      scratch_types=[
          pltpu.VMEM((B0_PER_W,), jnp.int32),
          pltpu.VMEM((CHUNK, 2 * MAXDEG), jnp.int32),
          pltpu.VMEM((CHUNK, 2 * MAXDEG), jnp.int32),
          pltpu.VMEM((SAMPLES2, CHUNK), jnp.int32),
          pltpu.VMEM((SAMPLES2, CHUNK), jnp.int32),
          pltpu.VMEM((SAMPLES1, CHUNK), jnp.int32),
          pltpu.VMEM((SAMPLES1, CHUNK), jnp.int32),
          pltpu.SemaphoreType.DMA,
          pltpu.SemaphoreType.DMA,
          pltpu.SemaphoreType.DMA,
          pltpu.SemaphoreType.DMA,
      ],
  )
  def body(x_h, adj_h, out0_h, out1_h,
           xs_v, rows0_v, rows1_v, out0t_v, idx1_v, stg0_v, stg1_v,
           gsem0, gsem1, osem0, osem1):
    wid = lax.axis_index("s") * 2 + lax.axis_index("c")
    lane = lax.iota(jnp.int32, 16)
    rowss = [rows0_v, rows1_v]
    stgs = [stg0_v, stg1_v]
    gsems = [gsem0, gsem1]
    osems = [osem0, osem1]

    pltpu.sync_copy(x_h.at[pl.ds(wid * B0_PER_W, B0_PER_W)], xs_v)
    pltpu.async_copy(adj_h.at[xs_v], rows0_v, gsem0).wait()

    def sel0(rb, carry):
      base = rb * 16
      rvec = base + lane
      for j in range(SAMPLES2):
        vals = plsc.load_gather(
            rows0_v, [rvec, jnp.full((16,), _PAD + _PERM0[j], jnp.int32)])
        out0t_v[j, pl.ds(base, 16)] = vals
        f = rvec * SAMPLES2 + j
        plsc.store_scatter(
            idx1_v,
            [lax.shift_right_logical(f, 7), lax.bitwise_and(f, 127)], vals)
      return carry

    lax.fori_loop(0, RGROUPS, sel0, 0)
    pltpu.async_copy(out0t_v, out0_h.at[:, pl.ds(wid * B0_PER_W, B0_PER_W)],
                     osem0).wait()

    out_copies = [None, None]
    gather_copies = [None, None]
    gather_copies[0] = pltpu.async_copy(
        adj_h.at[idx1_v.at[0]], rows0_v, gsem0)
    for c in range(N_CHUNKS):
      b = c % 2
      if c + 1 < N_CHUNKS:
        gather_copies[(c + 1) % 2] = pltpu.async_copy(
            adj_h.at[idx1_v.at[c + 1]], rowss[(c + 1) % 2],
            gsems[(c + 1) % 2])
      gather_copies[b].wait()
      if out_copies[b] is not None:
        out_copies[b].wait()
      rows_b, stg_b = rowss[b], stgs[b]

      def sel1(rb, carry):
        base = rb * 16
        rvec = base + lane
        for j in range(SAMPLES1):
          vals = plsc.load_gather(
              rows_b, [rvec, jnp.full((16,), _PAD + _PERM1[j], jnp.int32)])
          stg_b[j, pl.ds(base, 16)] = vals
        return carry

      lax.fori_loop(0, RGROUPS, sel1, 0)
      out_copies[b] = pltpu.async_copy(
          stgs[b],
          out1_h.at[:, pl.ds(wid * N_CHUNKS * CHUNK + c * CHUNK, CHUNK)],
          osems[b])
    out_copies[0].wait()
    out_copies[1].wait()

  return body(x, adj_p)


def kernel(x, adj_table):
  adj_p = jnp.pad(adj_table, ((0, 0), (_PAD, 0)))
  out0t, out1t = _sampler_kernel(x, adj_p)
  return (out0t.T, out1t.T)
